# Initial kernel scaffold; baseline (speedup 1.0000x reference)
import jax
import jax.numpy as jnp
from jax.experimental import pallas as pl


def kernel(x, time_emb, freq_emb, type_emb):
    raise NotImplementedError("write your pallas kernel here")


# speedup vs baseline: 2.0317x; 2.0317x over previous
import jax
import jax.numpy as jnp
from jax.experimental import pallas as pl

N_TIME = 128
N_FREQ = 32
D_MODEL = 128
N_ERP = 128
N_TFR = N_TIME * N_FREQ
N_TOKENS = N_ERP + N_TFR
BATCH = 16


def _body(x_ref, time_ref, freq_ref, type_ref, out_ref):
    t = time_ref[...]
    ty0 = type_ref[0:1, :]
    ty1 = type_ref[1:2, :]
    f = freq_ref[...]

    out_ref[0, :N_ERP, :] = x_ref[0, :N_ERP, :] + (t + ty0)

    xr = x_ref[0, N_ERP:, :].reshape(N_TIME, N_FREQ, D_MODEL)
    pe_tfr = xr + t[:, None, :] + f[None, :, :] + ty1[None, :, :]
    out_ref[0, N_ERP:, :] = pe_tfr.reshape(N_TFR, D_MODEL)


def kernel(x, time_emb, freq_emb, type_emb):
    return pl.pallas_call(
        _body,
        grid=(BATCH,),
        in_specs=[
            pl.BlockSpec((1, N_TOKENS, D_MODEL), lambda b: (b, 0, 0)),
            pl.BlockSpec((N_TIME, D_MODEL), lambda b: (0, 0)),
            pl.BlockSpec((N_FREQ, D_MODEL), lambda b: (0, 0)),
            pl.BlockSpec((2, D_MODEL), lambda b: (0, 0)),
        ],
        out_specs=pl.BlockSpec((1, N_TOKENS, D_MODEL), lambda b: (b, 0, 0)),
        out_shape=jax.ShapeDtypeStruct((BATCH, N_TOKENS, D_MODEL), jnp.float32),
    )(x, time_emb, freq_emb, type_emb)
